# baseline (speedup 1.0000x reference)
import jax
import jax.numpy as jnp
from jax.experimental import pallas as pl

N = 32768
D = 768
H = 4
P = 64
K1 = 8
K2 = 2
DH = D // H
DFF = 4 * D

ROWS = 1024
NEG = -1e30


def _fused_kernel(scal_ref, x_ref, bias_ref, scale_ref, W1_ref, b1_ref,
                  W2_ref, b2_ref, W3_ref, b3_ref, Wg_ref, proto_ref, Wo_ref,
                  out_ref, ti2_ref):
    sa = jax.nn.sigmoid(scal_ref[0, 0])
    sb = jax.nn.sigmoid(scal_ref[0, 1])
    sg = jax.nn.sigmoid(scal_ref[0, 2])

    xg = sg * x_ref[...]
    ssq = jnp.sum(xg * xg, axis=1, keepdims=True)
    rms = jnp.sqrt(ssq) * (D ** -0.5)
    xn = scale_ref[...] * (xg / (rms + 1e-8))
    logits = jnp.dot(xn, Wg_ref[...], preferred_element_type=jnp.float32)

    xnb = xn.astype(jnp.bfloat16)
    CH = DFF // H
    iota_col = jax.lax.broadcasted_iota(jnp.int32, (P, 1), 0).astype(jnp.float32)
    wfulls = []
    ti_parts = []
    ffn = None
    for h in range(H):
        c0 = h * CH
        h1c = jnp.dot(xnb, W1_ref[:, c0:c0 + CH],
                      preferred_element_type=jnp.float32) + b1_ref[:, c0:c0 + CH]
        h2c = jnp.dot(xnb, W2_ref[:, c0:c0 + CH],
                      preferred_element_type=jnp.float32) + b2_ref[:, c0:c0 + CH]
        hhc = (h1c * jax.nn.sigmoid(h1c)) * h2c
        f = jnp.dot(hhc.astype(jnp.bfloat16), W3_ref[c0:c0 + CH, :],
                    preferred_element_type=jnp.float32)
        ffn = f if ffn is None else ffn + f
        lh = logits[:, h * P:(h + 1) * P]
        biased = lh + bias_ref[...]
        work = biased
        for _ in range(K1):
            m = jnp.max(work, axis=1, keepdims=True)
            work = jnp.where(work == m, NEG, work)
        work2 = jnp.where(work == NEG, lh, NEG)
        ohs, vals, picks = [], [], []
        for _ in range(K2):
            m = jnp.max(work2, axis=1, keepdims=True)
            eq = work2 >= m - (jnp.abs(m) * 5e-7 + 1e-9)
            bsel = jnp.where(eq, biased, NEG)
            m2 = jnp.max(bsel, axis=1, keepdims=True)
            ohb = bsel == m2
            ohf = ohb.astype(jnp.float32)
            pick = jnp.dot(ohf, iota_col, preferred_element_type=jnp.float32)
            ohs.append(ohf)
            vals.append(m)
            picks.append(pick.astype(jnp.int32))
            work2 = jnp.where(ohb, NEG, work2)
        e = jnp.exp(vals[1] - vals[0])
        w0 = 1.0 / (1.0 + e)
        w1 = e * w0
        wfulls.append(w0 * ohs[0] + w1 * ohs[1])
        ti_parts.append(jnp.concatenate(picks, axis=1))

    wfull_all = jnp.concatenate(wfulls, axis=1)
    a_h = jnp.dot(wfull_all, proto_ref[...], preferred_element_type=jnp.float32)
    a = jnp.dot(a_h.astype(jnp.bfloat16), Wo_ref[...],
                preferred_element_type=jnp.float32)

    out_ref[...] = sa * (ffn + b3_ref[...]) + sb * a
    ti2_ref[...] = jnp.concatenate(ti_parts, axis=1)


def kernel(x, bias, scale, W1, b1, W2, b2, W3, b3, Wg, proto, Wo,
           alpha, beta, gamma, delta):
    scal = jnp.stack([alpha, beta, gamma, delta]).reshape(1, 4)
    Wg2 = Wg.reshape(D, H * P)
    eye = jnp.eye(H, dtype=proto.dtype)
    proto_bd = (eye[:, None, :, None] * proto[:, :, None, :]).reshape(H * P, D)
    out, ti2 = pl.pallas_call(
        _fused_kernel,
        grid=(N // ROWS,),
        in_specs=[
            pl.BlockSpec((1, 4), lambda i: (0, 0)),
            pl.BlockSpec((ROWS, D), lambda i: (i, 0)),
            pl.BlockSpec((1, P), lambda i: (0, 0)),
            pl.BlockSpec((1, D), lambda i: (0, 0)),
            pl.BlockSpec((D, DFF), lambda i: (0, 0)),
            pl.BlockSpec((1, DFF), lambda i: (0, 0)),
            pl.BlockSpec((D, DFF), lambda i: (0, 0)),
            pl.BlockSpec((1, DFF), lambda i: (0, 0)),
            pl.BlockSpec((DFF, D), lambda i: (0, 0)),
            pl.BlockSpec((1, D), lambda i: (0, 0)),
            pl.BlockSpec((D, H * P), lambda i: (0, 0)),
            pl.BlockSpec((H * P, D), lambda i: (0, 0)),
            pl.BlockSpec((D, D), lambda i: (0, 0)),
        ],
        out_specs=[
            pl.BlockSpec((ROWS, D), lambda i: (i, 0)),
            pl.BlockSpec((ROWS, H * K2), lambda i: (i, 0)),
        ],
        out_shape=[
            jax.ShapeDtypeStruct((N, D), jnp.float32),
            jax.ShapeDtypeStruct((N, H * K2), jnp.int32),
        ],
    )(scal, x, bias.reshape(1, P), scale.reshape(1, D),
      W1.astype(jnp.bfloat16), b1.reshape(1, DFF),
      W2.astype(jnp.bfloat16), b2.reshape(1, DFF),
      W3.astype(jnp.bfloat16), b3.reshape(1, D),
      Wg2, proto_bd, Wo.astype(jnp.bfloat16))
    return out, ti2.reshape(N, H, K2)


# speedup vs baseline: 1.0583x; 1.0583x over previous
import jax
import jax.numpy as jnp
from jax.experimental import pallas as pl

N = 32768
D = 768
H = 4
P = 64
K1 = 8
K2 = 2
DH = D // H
DFF = 4 * D

ROWS = 1024
NEG = -1e30


def _fused_kernel(scal_ref, x_ref, bias_ref, scale_ref, W1_ref, b1_ref,
                  W2_ref, b2_ref, W3_ref, b3_ref, Wg_ref, proto_ref, Wo_ref,
                  out_ref, ti2_ref):
    sa = jax.nn.sigmoid(scal_ref[0, 0])
    sb = jax.nn.sigmoid(scal_ref[0, 1])
    sg = jax.nn.sigmoid(scal_ref[0, 2])

    xg = sg * x_ref[...]
    ssq = jnp.sum(xg * xg, axis=1, keepdims=True)
    rms = jnp.sqrt(ssq) * (D ** -0.5)
    xn = scale_ref[...] * (xg / (rms + 1e-8))
    logits = jnp.dot(xn, Wg_ref[...], preferred_element_type=jnp.float32)

    xnb = xn.astype(jnp.bfloat16)
    CH = DFF // H
    iota_col = jax.lax.broadcasted_iota(jnp.int32, (P, 1), 0).astype(jnp.float32)
    wfulls = []
    ti_parts = []
    ffn = None
    for h in range(H):
        c0 = h * CH
        h1c = jnp.dot(xnb, W1_ref[:, c0:c0 + CH],
                      preferred_element_type=jnp.float32) + b1_ref[:, c0:c0 + CH]
        h2c = jnp.dot(xnb, W2_ref[:, c0:c0 + CH],
                      preferred_element_type=jnp.float32) + b2_ref[:, c0:c0 + CH]
        hhc = (h1c * jax.nn.sigmoid(h1c)) * h2c
        f = jnp.dot(hhc.astype(jnp.bfloat16), W3_ref[c0:c0 + CH, :],
                    preferred_element_type=jnp.float32)
        ffn = f if ffn is None else ffn + f
        lh = logits[:, h * P:(h + 1) * P]
        biased = lh + bias_ref[...]
        work = biased
        for _ in range(K1):
            m = jnp.max(work, axis=1, keepdims=True)
            work = jnp.where(work == m, NEG, work)
        work2 = jnp.where(work == NEG, lh, NEG)
        ohs, vals, picks = [], [], []
        for _ in range(K2):
            m = jnp.max(work2, axis=1, keepdims=True)
            eq = work2 >= m - (jnp.abs(m) * 5e-7 + 1e-9)
            bsel = jnp.where(eq, biased, NEG)
            m2 = jnp.max(bsel, axis=1, keepdims=True)
            ohb = bsel == m2
            ohf = ohb.astype(jnp.float32)
            pick = jnp.dot(ohf, iota_col, preferred_element_type=jnp.float32)
            ohs.append(ohf)
            vals.append(m)
            picks.append(pick.astype(jnp.int32))
            work2 = jnp.where(ohb, NEG, work2)
        e = jnp.exp(vals[1] - vals[0])
        w0 = 1.0 / (1.0 + e)
        w1 = e * w0
        wfull = w0 * ohs[0] + w1 * ohs[1]
        wfulls.append(jnp.dot(wfull, proto_ref[h],
                              preferred_element_type=jnp.float32))
        ti_parts.append(jnp.concatenate(picks, axis=1))

    a_h = jnp.concatenate(wfulls, axis=1)
    a = jnp.dot(a_h.astype(jnp.bfloat16), Wo_ref[...],
                preferred_element_type=jnp.float32)

    out_ref[...] = sa * (ffn + b3_ref[...]) + sb * a
    ti2_ref[...] = jnp.concatenate(ti_parts, axis=1)


def kernel(x, bias, scale, W1, b1, W2, b2, W3, b3, Wg, proto, Wo,
           alpha, beta, gamma, delta):
    scal = jnp.stack([alpha, beta, gamma, delta]).reshape(1, 4)
    Wg2 = Wg.reshape(D, H * P)
    out, ti2 = pl.pallas_call(
        _fused_kernel,
        grid=(N // ROWS,),
        in_specs=[
            pl.BlockSpec((1, 4), lambda i: (0, 0)),
            pl.BlockSpec((ROWS, D), lambda i: (i, 0)),
            pl.BlockSpec((1, P), lambda i: (0, 0)),
            pl.BlockSpec((1, D), lambda i: (0, 0)),
            pl.BlockSpec((D, DFF), lambda i: (0, 0)),
            pl.BlockSpec((1, DFF), lambda i: (0, 0)),
            pl.BlockSpec((D, DFF), lambda i: (0, 0)),
            pl.BlockSpec((1, DFF), lambda i: (0, 0)),
            pl.BlockSpec((DFF, D), lambda i: (0, 0)),
            pl.BlockSpec((1, D), lambda i: (0, 0)),
            pl.BlockSpec((D, H * P), lambda i: (0, 0)),
            pl.BlockSpec((H, P, DH), lambda i: (0, 0, 0)),
            pl.BlockSpec((D, D), lambda i: (0, 0)),
        ],
        out_specs=[
            pl.BlockSpec((ROWS, D), lambda i: (i, 0)),
            pl.BlockSpec((ROWS, H * K2), lambda i: (i, 0)),
        ],
        out_shape=[
            jax.ShapeDtypeStruct((N, D), jnp.float32),
            jax.ShapeDtypeStruct((N, H * K2), jnp.int32),
        ],
    )(scal, x, bias.reshape(1, P), scale.reshape(1, D),
      W1.astype(jnp.bfloat16), b1.reshape(1, DFF),
      W2.astype(jnp.bfloat16), b2.reshape(1, DFF),
      W3.astype(jnp.bfloat16), b3.reshape(1, D),
      Wg2, proto, Wo.astype(jnp.bfloat16))
    return out, ti2.reshape(N, H, K2)


# speedup vs baseline: 1.0857x; 1.0259x over previous
import jax
import jax.numpy as jnp
from jax.experimental import pallas as pl

N = 32768
D = 768
H = 4
P = 64
K1 = 8
K2 = 2
DH = D // H
DFF = 4 * D

ROWS = 1024
NEG = -1e30


def _fused_kernel(scal_ref, x_ref, bias_ref, scale_ref, W1_ref, b1_ref,
                  W2_ref, b2_ref, W3_ref, b3_ref, Wg_ref, proto_ref, Wo_ref,
                  out_ref, ti2_ref):
    sa = jax.nn.sigmoid(scal_ref[0, 0])
    sb = jax.nn.sigmoid(scal_ref[0, 1])
    sg = jax.nn.sigmoid(scal_ref[0, 2])

    xg = sg * x_ref[...]
    ssq = jnp.sum(xg * xg, axis=1, keepdims=True)
    rms = jnp.sqrt(ssq) * (D ** -0.5)
    xn = scale_ref[...] * (xg / (rms + 1e-8))

    xnb = xn.astype(jnp.bfloat16)
    CH = DFF // H
    logitsT = jax.lax.dot_general(Wg_ref[...], xn, (((0,), (1,)), ((), ())),
                                  preferred_element_type=jnp.float32)
    bias_col = bias_ref[...]
    iota_row = jax.lax.broadcasted_iota(jnp.int32, (1, P), 1).astype(jnp.float32)
    aT_parts = []
    ti_parts = []
    ffn = None
    for h in range(H):
        c0 = h * CH
        h1c = jnp.dot(xnb, W1_ref[:, c0:c0 + CH],
                      preferred_element_type=jnp.float32) + b1_ref[:, c0:c0 + CH]
        h2c = jnp.dot(xnb, W2_ref[:, c0:c0 + CH],
                      preferred_element_type=jnp.float32) + b2_ref[:, c0:c0 + CH]
        hhc = (h1c * jax.nn.sigmoid(h1c)) * h2c
        f = jnp.dot(hhc.astype(jnp.bfloat16), W3_ref[c0:c0 + CH, :],
                    preferred_element_type=jnp.float32)
        ffn = f if ffn is None else ffn + f
        lh = logitsT[h * P:(h + 1) * P, :]
        biased = lh + bias_col
        work = biased
        for _ in range(K1):
            m = jnp.max(work, axis=0, keepdims=True)
            work = jnp.where(work == m, NEG, work)
        work2 = jnp.where(work == NEG, lh, NEG)
        ohs, vals, picks = [], [], []
        for _ in range(K2):
            m = jnp.max(work2, axis=0, keepdims=True)
            eq = work2 >= m - (jnp.abs(m) * 5e-7 + 1e-9)
            bsel = jnp.where(eq, biased, NEG)
            m2 = jnp.max(bsel, axis=0, keepdims=True)
            ohb = bsel == m2
            ohf = ohb.astype(jnp.float32)
            pick = jnp.dot(iota_row, ohf, preferred_element_type=jnp.float32)
            ohs.append(ohf)
            vals.append(m)
            picks.append(pick.astype(jnp.int32))
            work2 = jnp.where(ohb, NEG, work2)
        e = jnp.exp(vals[1] - vals[0])
        w0 = 1.0 / (1.0 + e)
        w1 = e * w0
        wfullT = w0 * ohs[0] + w1 * ohs[1]
        aT_parts.append(jax.lax.dot_general(
            proto_ref[h], wfullT, (((0,), (0,)), ((), ())),
            preferred_element_type=jnp.float32))
        ti_parts.append(jnp.concatenate(picks, axis=0))

    a_hT = jnp.concatenate(aT_parts, axis=0)
    a = jax.lax.dot_general(a_hT.astype(jnp.bfloat16), Wo_ref[...],
                            (((0,), (0,)), ((), ())),
                            preferred_element_type=jnp.float32)

    out_ref[...] = sa * (ffn + b3_ref[...]) + sb * a
    ti2_ref[...] = jnp.concatenate(ti_parts, axis=0)


def kernel(x, bias, scale, W1, b1, W2, b2, W3, b3, Wg, proto, Wo,
           alpha, beta, gamma, delta):
    scal = jnp.stack([alpha, beta, gamma, delta]).reshape(1, 4)
    Wg2 = Wg.reshape(D, H * P)
    out, ti2 = pl.pallas_call(
        _fused_kernel,
        grid=(N // ROWS,),
        in_specs=[
            pl.BlockSpec((1, 4), lambda i: (0, 0)),
            pl.BlockSpec((ROWS, D), lambda i: (i, 0)),
            pl.BlockSpec((P, 1), lambda i: (0, 0)),
            pl.BlockSpec((1, D), lambda i: (0, 0)),
            pl.BlockSpec((D, DFF), lambda i: (0, 0)),
            pl.BlockSpec((1, DFF), lambda i: (0, 0)),
            pl.BlockSpec((D, DFF), lambda i: (0, 0)),
            pl.BlockSpec((1, DFF), lambda i: (0, 0)),
            pl.BlockSpec((DFF, D), lambda i: (0, 0)),
            pl.BlockSpec((1, D), lambda i: (0, 0)),
            pl.BlockSpec((D, H * P), lambda i: (0, 0)),
            pl.BlockSpec((H, P, DH), lambda i: (0, 0, 0)),
            pl.BlockSpec((D, D), lambda i: (0, 0)),
        ],
        out_specs=[
            pl.BlockSpec((ROWS, D), lambda i: (i, 0)),
            pl.BlockSpec((H * K2, ROWS), lambda i: (0, i)),
        ],
        out_shape=[
            jax.ShapeDtypeStruct((N, D), jnp.float32),
            jax.ShapeDtypeStruct((H * K2, N), jnp.int32),
        ],
    )(scal, x, bias.reshape(P, 1), scale.reshape(1, D),
      W1.astype(jnp.bfloat16), b1.reshape(1, DFF),
      W2.astype(jnp.bfloat16), b2.reshape(1, DFF),
      W3.astype(jnp.bfloat16), b3.reshape(1, D),
      Wg2, proto, Wo.astype(jnp.bfloat16))
    return out, ti2.T.reshape(N, H, K2)
